# Initial kernel scaffold; baseline (speedup 1.0000x reference)
import jax
import jax.numpy as jnp
from jax.experimental import pallas as pl


def kernel(input_ids, sent_position_ids, sent_coordinate_list, word_emb, pos_emb, Wr, gamma, beta):
    raise NotImplementedError("write your pallas kernel here")


# speedup vs baseline: 1.7358x; 1.7358x over previous
import functools

import jax
import jax.numpy as jnp
from jax import lax
from jax.experimental import pallas as pl
from jax.experimental.pallas import tpu as pltpu
from jax.experimental.pallas import tpu_sc as plsc

VOCAB = 100000
H = 768
MAXPOS = 4096
EPS = 1e-12
DIV = float(jnp.sqrt(jnp.float32(H)))

NC = 2
NS = 16
NW = NC * NS
NTOK = 16384
TOK_PER_W = NTOK // NW
CHUNK = 64
NCHUNK = TOK_PER_W // CHUNK
LANES = 16
HVECS = H // LANES


def _sc_gather_sum(word_emb, pos_emb, ids, pos_ids):
    mesh = plsc.VectorSubcoreMesh(core_axis_name="c", subcore_axis_name="s")

    @functools.partial(
        pl.kernel,
        out_type=jax.ShapeDtypeStruct((NTOK, H), jnp.float32),
        mesh=mesh,
        scratch_types=[
            pltpu.VMEM((NCHUNK, CHUNK), jnp.int32),
            pltpu.VMEM((NCHUNK, CHUNK), jnp.int32),
            pltpu.VMEM((CHUNK, H), jnp.float32),
            pltpu.VMEM((CHUNK, H), jnp.float32),
            pltpu.SemaphoreType.DMA,
            pltpu.SemaphoreType.DMA,
        ],
    )
    def k(word_hbm, pos_hbm, ids_hbm, pids_hbm, out_hbm,
          idx_v, pidx_v, bufw, bufp, semw, semp):
        wid = lax.axis_index("s") * NC + lax.axis_index("c")
        base = wid * TOK_PER_W
        pltpu.sync_copy(ids_hbm.at[wid], idx_v)
        pltpu.sync_copy(pids_hbm.at[wid], pidx_v)

        def chunk_body(j, _):
            cw = pltpu.async_copy(word_hbm.at[idx_v.at[j]], bufw, semw)
            cp = pltpu.async_copy(pos_hbm.at[pidx_v.at[j]], bufp, semp)
            cw.wait()
            cp.wait()

            def row_body(i, _):
                for kk in range(HVECS):
                    sl = pl.ds(kk * LANES, LANES)
                    bufw[i, sl] = bufw[i, sl] + bufp[i, sl]
                return _

            lax.fori_loop(0, CHUNK, row_body, None)
            pltpu.sync_copy(bufw, out_hbm.at[pl.ds(base + j * CHUNK, CHUNK)])
            return _

        lax.fori_loop(0, NCHUNK, chunk_body, None)

    return k(word_emb, pos_emb, ids, pos_ids)


def _tc_finish_body(sum_ref, c_ref, wrt_ref, g_ref, b_ref, out_ref):
    x = sum_ref[...]
    c = c_ref[...]
    w = wrt_ref[...]
    theta = c[:, 0:1] * w[0:1, :] + c[:, 1:2] * w[1:2, :]
    pe = jnp.concatenate([jnp.sin(theta), jnp.cos(theta)], axis=-1) * (1.0 / DIV)
    x = x + pe
    mean = jnp.mean(x, axis=-1, keepdims=True)
    xc = x - mean
    var = jnp.mean(xc * xc, axis=-1, keepdims=True)
    y = xc * lax.rsqrt(var + EPS)
    out_ref[...] = y * g_ref[...] + b_ref[...]


def _tc_finish(summed, coords, wrt, gamma, beta):
    TB = 512
    grid = (NTOK // TB,)
    return pl.pallas_call(
        _tc_finish_body,
        grid=grid,
        in_specs=[
            pl.BlockSpec((TB, H), lambda i: (i, 0)),
            pl.BlockSpec((TB, 2), lambda i: (i, 0)),
            pl.BlockSpec((2, H // 2), lambda i: (0, 0)),
            pl.BlockSpec((1, H), lambda i: (0, 0)),
            pl.BlockSpec((1, H), lambda i: (0, 0)),
        ],
        out_specs=pl.BlockSpec((TB, H), lambda i: (i, 0)),
        out_shape=jax.ShapeDtypeStruct((NTOK, H), jnp.float32),
    )(summed, coords, wrt, gamma, beta)


def kernel(input_ids, sent_position_ids, sent_coordinate_list, word_emb,
           pos_emb, Wr, gamma, beta):
    B, S = input_ids.shape
    ids = input_ids.astype(jnp.int32).reshape(NW, NCHUNK, CHUNK)
    pids = sent_position_ids.astype(jnp.int32).reshape(NW, NCHUNK, CHUNK)
    summed = _sc_gather_sum(word_emb, pos_emb, ids, pids)
    coords = sent_coordinate_list.reshape(NTOK, 2)
    out = _tc_finish(summed, coords, Wr.T, gamma.reshape(1, H),
                     beta.reshape(1, H))
    return out.reshape(B, S, H)


# speedup vs baseline: 2.2770x; 1.3118x over previous
import functools
import math

import jax
import jax.numpy as jnp
import numpy as np
from jax import lax
from jax.experimental import pallas as pl
from jax.experimental.pallas import tpu as pltpu
from jax.experimental.pallas import tpu_sc as plsc

VOCAB = 100000
H = 768
MAXPOS = 4096
EPS = 1e-12
DIV = math.sqrt(float(H))

NC = 2
NS = 16
NW = NC * NS
NTOK = 16384
TOK_PER_W = NTOK // NW
CHUNK = 64
NCHUNK = TOK_PER_W // CHUNK
LANES = 16
HVECS = H // LANES


def _sc_gather_sum(word_emb, pos_emb, ids, pos_ids):
    mesh = plsc.VectorSubcoreMesh(core_axis_name="c", subcore_axis_name="s")

    @functools.partial(
        pl.kernel,
        out_type=jax.ShapeDtypeStruct((NTOK, H), jnp.float32),
        mesh=mesh,
        scratch_types=[
            pltpu.VMEM((NCHUNK, CHUNK), jnp.int32),
            pltpu.VMEM((NCHUNK, CHUNK), jnp.int32),
            pltpu.VMEM((CHUNK, H), jnp.float32),
            pltpu.VMEM((CHUNK, H), jnp.float32),
            pltpu.SemaphoreType.DMA,
            pltpu.SemaphoreType.DMA,
        ],
    )
    def k(word_hbm, pos_hbm, ids_hbm, pids_hbm, out_hbm,
          idx_v, pidx_v, bufw, bufp, semw, semp):
        wid = lax.axis_index("s") * NC + lax.axis_index("c")
        base = wid * TOK_PER_W
        pltpu.sync_copy(ids_hbm.at[wid], idx_v)
        pltpu.sync_copy(pids_hbm.at[wid], pidx_v)

        def chunk_body(j, _):
            cw = pltpu.async_copy(word_hbm.at[idx_v.at[j]], bufw, semw)
            cp = pltpu.async_copy(pos_hbm.at[pidx_v.at[j]], bufp, semp)
            cw.wait()
            cp.wait()

            def row_body(i, _):
                for kk in range(HVECS):
                    sl = pl.ds(kk * LANES, LANES)
                    bufw[i, sl] = bufw[i, sl] + bufp[i, sl]
                return _

            lax.fori_loop(0, CHUNK, row_body, None)
            pltpu.sync_copy(bufw, out_hbm.at[pl.ds(base + j * CHUNK, CHUNK)])
            return _

        lax.fori_loop(0, NCHUNK, chunk_body, None)

    return k(word_emb, pos_emb, ids, pos_ids)


_INV_2PI = float(np.float32(1.0 / (2.0 * math.pi)))
_PI2_HI = float(np.float32(2.0 * math.pi))
_PI2_LO = 2.0 * math.pi - _PI2_HI
_SIN_C = (0.9999972899502265, -0.16665146113621815, 0.008319843694968633,
          -0.0001942418188105692, 2.2248881392794573e-06)
_COS_C = (0.9999994437075935, -0.49999558228580177, 0.04166103351910408,
          -0.0013862749961056388, 2.4253229890178196e-05,
          -2.2194129828401188e-07)


def _fast_sincos(theta):
    r = theta * _INV_2PI
    half = jnp.where(r >= 0, jnp.float32(0.5), jnp.float32(-0.5))
    k = (r + half).astype(jnp.int32).astype(jnp.float32)
    m = theta - k * _PI2_HI
    m = m - k * _PI2_LO
    u = m * m
    ps = jnp.float32(_SIN_C[-1])
    for a in _SIN_C[-2::-1]:
        ps = ps * u + jnp.float32(a)
    pc = jnp.float32(_COS_C[-1])
    for a in _COS_C[-2::-1]:
        pc = pc * u + jnp.float32(a)
    return m * ps, pc


def _tc_finish_body(sum_ref, c_ref, wrt_ref, g_ref, b_ref, out_ref):
    HH = H // 2
    c = c_ref[...]
    w = wrt_ref[...]
    theta = c[:, 0:1] * w[0:1, :] + c[:, 1:2] * w[1:2, :]
    s, co = _fast_sincos(theta)
    inv = jnp.float32(1.0 / DIV)
    x1 = sum_ref[:, :HH] + s * inv
    x2 = sum_ref[:, HH:] + co * inv
    tot = jnp.sum(x1, axis=-1, keepdims=True) + jnp.sum(x2, axis=-1, keepdims=True)
    mean = tot * jnp.float32(1.0 / H)
    xc1 = x1 - mean
    xc2 = x2 - mean
    ss = (jnp.sum(xc1 * xc1, axis=-1, keepdims=True)
          + jnp.sum(xc2 * xc2, axis=-1, keepdims=True))
    rstd = lax.rsqrt(ss * jnp.float32(1.0 / H) + EPS)
    out_ref[:, :HH] = xc1 * rstd * g_ref[:, :HH] + b_ref[:, :HH]
    out_ref[:, HH:] = xc2 * rstd * g_ref[:, HH:] + b_ref[:, HH:]


def _tc_finish(summed, coords, wrt, gamma, beta):
    TB = 512
    grid = (NTOK // TB,)
    return pl.pallas_call(
        _tc_finish_body,
        grid=grid,
        in_specs=[
            pl.BlockSpec((TB, H), lambda i: (i, 0)),
            pl.BlockSpec((TB, 2), lambda i: (i, 0)),
            pl.BlockSpec((2, H // 2), lambda i: (0, 0)),
            pl.BlockSpec((1, H), lambda i: (0, 0)),
            pl.BlockSpec((1, H), lambda i: (0, 0)),
        ],
        out_specs=pl.BlockSpec((TB, H), lambda i: (i, 0)),
        out_shape=jax.ShapeDtypeStruct((NTOK, H), jnp.float32),
    )(summed, coords, wrt, gamma, beta)


def kernel(input_ids, sent_position_ids, sent_coordinate_list, word_emb,
           pos_emb, Wr, gamma, beta):
    B, S = input_ids.shape
    ids = input_ids.astype(jnp.int32).reshape(NW, NCHUNK, CHUNK)
    pids = sent_position_ids.astype(jnp.int32).reshape(NW, NCHUNK, CHUNK)
    summed = _sc_gather_sum(word_emb, pos_emb, ids, pids)
    coords = sent_coordinate_list.reshape(NTOK, 2)
    out = _tc_finish(summed, coords, Wr.T, gamma.reshape(1, H),
                     beta.reshape(1, H))
    return out.reshape(B, S, H)


# speedup vs baseline: 2.8125x; 1.2352x over previous
import functools
import math

import jax
import jax.numpy as jnp
import numpy as np
from jax import lax
from jax.experimental import pallas as pl
from jax.experimental.pallas import tpu as pltpu
from jax.experimental.pallas import tpu_sc as plsc

VOCAB = 100000
H = 768
MAXPOS = 4096
EPS = 1e-12
DIV = math.sqrt(float(H))

NC = 2
NS = 16
NW = NC * NS
NTOK = 16384
TOK_PER_W = NTOK // NW
CHUNK = 16
NCHUNK = TOK_PER_W // CHUNK
NBUF = 4
LANES = 16
HVECS = H // LANES


def _sc_gather_sum(word_emb, pos_emb, ids, pos_ids):
    mesh = plsc.VectorSubcoreMesh(core_axis_name="c", subcore_axis_name="s")

    @functools.partial(
        pl.kernel,
        out_type=jax.ShapeDtypeStruct((NTOK, H), jnp.float32),
        mesh=mesh,
        scratch_types=[
            pltpu.VMEM((NCHUNK, CHUNK), jnp.int32),
            pltpu.VMEM((NCHUNK, CHUNK), jnp.int32),
        ] + [pltpu.VMEM((CHUNK, H), jnp.float32) for _ in range(2 * NBUF)] + [
            pltpu.SemaphoreType.DMA,
        ] * (3 * NBUF),
    )
    def k(word_hbm, pos_hbm, ids_hbm, pids_hbm, out_hbm,
          idx_v, pidx_v, *rest):
        bw = rest[0:NBUF]
        bp = rest[NBUF:2 * NBUF]
        semw = rest[2 * NBUF:3 * NBUF]
        semp = rest[3 * NBUF:4 * NBUF]
        semo = rest[4 * NBUF:5 * NBUF]
        wid = lax.axis_index("s") * NC + lax.axis_index("c")
        base = wid * TOK_PER_W
        pltpu.sync_copy(ids_hbm.at[wid], idx_v)
        pltpu.sync_copy(pids_hbm.at[wid], pidx_v)

        def start_gather(j, b):
            pltpu.async_copy(word_hbm.at[idx_v.at[j]], bw[b], semw[b])
            pltpu.async_copy(pos_hbm.at[pidx_v.at[j]], bp[b], semp[b])

        def wait_gather(b):
            pltpu.make_async_copy(word_hbm.at[idx_v.at[0]], bw[b], semw[b]).wait()
            pltpu.make_async_copy(pos_hbm.at[pidx_v.at[0]], bp[b], semp[b]).wait()

        def wait_write(b):
            pltpu.make_async_copy(bw[b], out_hbm.at[pl.ds(0, CHUNK)], semo[b]).wait()

        for j in range(NBUF - 1):
            start_gather(j, j)

        def outer(jj, _):
            for b4 in range(NBUF):
                j = jj * NBUF + b4

                b_prev = (b4 - 1) % NBUF
                b_next = (b4 + NBUF - 1) % NBUF

                @pl.when(j >= 1)
                def _():
                    wait_write(b_prev)

                @pl.when(j + NBUF - 1 < NCHUNK)
                def _():
                    start_gather(j + NBUF - 1, b_next)

                wait_gather(b4)
                bwb = bw[b4]
                bpb = bp[b4]

                def row_body(i, carry):
                    for kk in range(HVECS):
                        sl = pl.ds(kk * LANES, LANES)
                        bwb[i, sl] = bwb[i, sl] + bpb[i, sl]
                    return carry

                lax.fori_loop(0, CHUNK, row_body, 0)
                pltpu.async_copy(
                    bwb, out_hbm.at[pl.ds(base + j * CHUNK, CHUNK)], semo[b4])
            return 0

        lax.fori_loop(0, NCHUNK // NBUF, outer, 0)
        wait_write((NCHUNK - 1) % NBUF)

    return k(word_emb, pos_emb, ids, pos_ids)


_INV_2PI = float(np.float32(1.0 / (2.0 * math.pi)))
_PI2_HI = float(np.float32(2.0 * math.pi))
_PI2_LO = 2.0 * math.pi - _PI2_HI
_SIN_C = (0.9999972899502265, -0.16665146113621815, 0.008319843694968633,
          -0.0001942418188105692, 2.2248881392794573e-06)
_COS_C = (0.9999994437075935, -0.49999558228580177, 0.04166103351910408,
          -0.0013862749961056388, 2.4253229890178196e-05,
          -2.2194129828401188e-07)
_SIN_CS = tuple(a / DIV for a in _SIN_C)
_COS_CS = tuple(a / DIV for a in _COS_C)


def _fast_sincos(theta, sin_c=_SIN_C, cos_c=_COS_C):
    r = theta * _INV_2PI
    half = jnp.where(r >= 0, jnp.float32(0.5), jnp.float32(-0.5))
    k = (r + half).astype(jnp.int32).astype(jnp.float32)
    m = theta - k * _PI2_HI
    m = m - k * _PI2_LO
    u = m * m
    ps = jnp.float32(sin_c[-1])
    for a in sin_c[-2::-1]:
        ps = ps * u + jnp.float32(a)
    pc = jnp.float32(cos_c[-1])
    for a in cos_c[-2::-1]:
        pc = pc * u + jnp.float32(a)
    return m * ps, pc


def _tc_finish_body(sum_ref, c_ref, wrt_ref, g_ref, b_ref, out_ref):
    HH = H // 2
    c = c_ref[...]
    w = wrt_ref[...]
    theta = lax.dot_general(c, w, (((1,), (0,)), ((), ())),
                            preferred_element_type=jnp.float32)
    s, co = _fast_sincos(theta, _SIN_CS, _COS_CS)
    x1 = sum_ref[:, :HH] + s
    x2 = sum_ref[:, HH:] + co
    tot = jnp.sum(x1, axis=-1, keepdims=True) + jnp.sum(x2, axis=-1, keepdims=True)
    mean = tot * jnp.float32(1.0 / H)
    xc1 = x1 - mean
    xc2 = x2 - mean
    ss = (jnp.sum(xc1 * xc1, axis=-1, keepdims=True)
          + jnp.sum(xc2 * xc2, axis=-1, keepdims=True))
    rstd = lax.rsqrt(ss * jnp.float32(1.0 / H) + EPS)
    out_ref[:, :HH] = xc1 * rstd * g_ref[:, :HH] + b_ref[:, :HH]
    out_ref[:, HH:] = xc2 * rstd * g_ref[:, HH:] + b_ref[:, HH:]


def _tc_finish(summed, coords, wrt, gamma, beta):
    TB = 512
    grid = (NTOK // TB,)
    return pl.pallas_call(
        _tc_finish_body,
        grid=grid,
        in_specs=[
            pl.BlockSpec((TB, H), lambda i: (i, 0)),
            pl.BlockSpec((TB, 2), lambda i: (i, 0)),
            pl.BlockSpec((2, H // 2), lambda i: (0, 0)),
            pl.BlockSpec((1, H), lambda i: (0, 0)),
            pl.BlockSpec((1, H), lambda i: (0, 0)),
        ],
        out_specs=pl.BlockSpec((TB, H), lambda i: (i, 0)),
        out_shape=jax.ShapeDtypeStruct((NTOK, H), jnp.float32),
    )(summed, coords, wrt, gamma, beta)


def kernel(input_ids, sent_position_ids, sent_coordinate_list, word_emb,
           pos_emb, Wr, gamma, beta):
    B, S = input_ids.shape
    ids = input_ids.astype(jnp.int32).reshape(NW, NCHUNK, CHUNK)
    pids = sent_position_ids.astype(jnp.int32).reshape(NW, NCHUNK, CHUNK)
    summed = _sc_gather_sum(word_emb, pos_emb, ids, pids)
    coords = sent_coordinate_list.reshape(NTOK, 2)
    out = _tc_finish(summed, coords, Wr.T, gamma.reshape(1, H),
                     beta.reshape(1, H))
    return out.reshape(B, S, H)
